# baseline (speedup 1.0000x reference)
import functools

import jax
import jax.numpy as jnp
from jax.experimental import pallas as pl

D_MODEL = 4096
N_EXP = 64
K = 8
T_BLK = 1024


def _router_body(x_ref, wt_ref, idx_ref, gate_ref, logits_ref):
    lt = jax.lax.dot_general(
        wt_ref[...], x_ref[...], (((1,), (1,)), ((), ())),
        preferred_element_type=jnp.float32)
    logits_ref[...] = lt.T
    iota = jax.lax.broadcasted_iota(jnp.int32, lt.shape, 0).astype(jnp.float32)
    cur = lt
    vals = []
    idxs = []
    for _ in range(K):
        m = jnp.max(cur, axis=0, keepdims=True)
        amax = jnp.min(
            jnp.where(cur == m, iota, jnp.float32(N_EXP)), axis=0, keepdims=True
        )
        vals.append(m)
        idxs.append(amax)
        cur = jnp.where(iota == amax, -jnp.inf, cur)

    tv = jnp.concatenate(vals, axis=0)
    ti = jnp.concatenate(idxs, axis=0)
    ev = jnp.exp(tv - tv[0:1, :])
    g = ev / jnp.sum(ev, axis=0, keepdims=True)
    gate_ref[...] = g.T
    idx_ref[...] = ti.T.astype(jnp.int32)


@jax.jit
def kernel(x, router_weights):
    b, s, d = x.shape
    n_tok = b * s
    x2 = x.reshape(n_tok, d)
    wt = router_weights

    grid = (n_tok // T_BLK,)
    idx_out, gates, logits = pl.pallas_call(
        _router_body,
        grid=grid,
        in_specs=[
            pl.BlockSpec((T_BLK, d), lambda i: (i, 0)),
            pl.BlockSpec((N_EXP, d), lambda i: (0, 0)),
        ],
        out_specs=[
            pl.BlockSpec((T_BLK, K), lambda i: (i, 0)),
            pl.BlockSpec((T_BLK, K), lambda i: (i, 0)),
            pl.BlockSpec((T_BLK, N_EXP), lambda i: (i, 0)),
        ],
        out_shape=[
            jax.ShapeDtypeStruct((n_tok, K), jnp.int32),
            jax.ShapeDtypeStruct((n_tok, K), jnp.float32),
            jax.ShapeDtypeStruct((n_tok, N_EXP), jnp.float32),
        ],
    )(x2, wt)

    return (
        idx_out.reshape(b, s, K),
        gates.reshape(b, s, K),
        logits.reshape(b, s, N_EXP),
    )
